# baseline (speedup 1.0000x reference)
import functools

import jax
import jax.numpy as jnp
from jax import lax
from jax.experimental import pallas as pl
from jax.experimental.pallas import tpu as pltpu
from jax.experimental.pallas import tpu_sc as plsc

_L = 16
_TILE = 128
_NBUF = 8
_NHALF = 1


@functools.lru_cache(maxsize=None)
def _build(seq_len: int, vocab: int, dim: int):
    info = plsc.get_sparse_core_info()
    nc, ns = info.num_cores, info.num_subcores
    nw = nc * ns
    nrow = dim // 8
    nunits = (seq_len // _TILE) * nrow
    assert nunits % nw == 0
    upw = nunits // nw
    ngroups = _TILE // _L

    mesh = plsc.VectorSubcoreMesh(core_axis_name="c", subcore_axis_name="s")

    @functools.partial(
        pl.kernel,
        mesh=mesh,
        out_type=jax.ShapeDtypeStruct((dim, seq_len), jnp.float32),
        scratch_types=[
            pltpu.VMEM((_TILE,), jnp.int32),
            pltpu.VMEM((2 * _L * 8, _TILE), jnp.float32),
            pltpu.VMEM((8, _TILE), jnp.float32),
            pltpu.VMEM((8, _TILE), jnp.float32),
            [pltpu.SemaphoreType.DMA] * 2,
            pltpu.SemaphoreType.DMA,
        ],
        compiler_params=pltpu.CompilerParams(needs_layout_passes=False),
    )
    def emb(x_hbm, wt_hbm, pt_hbm, out_hbm, idx_v, ring_v, buf_v, pos_v, sems, psem):
        wid = lax.axis_index("s") * nc + lax.axis_index("c")
        iota = lax.iota(jnp.int32, _L)
        half = iota < jnp.int32(8)
        sub = iota & jnp.int32(7)

        def fire_group(g, r8, par):
            vec = idx_v[pl.ds(g * _L, _L)]
            t128 = vec & jnp.int32(-_TILE)
            for k in range(_L):
                tk = pl.multiple_of(t128[k], _TILE)
                pltpu.async_copy(
                    wt_hbm.at[pl.ds(r8, 8), pl.ds(tk, _TILE)],
                    ring_v.at[pl.ds((par * _L + k) * 8, 8), :],
                    sems[par],
                )

        def unit(u):
            gu = wid * upw + u
            blk = gu // nrow
            row = gu % nrow
            r8 = pl.multiple_of(row * 8, 8)
            cbase = pl.multiple_of(blk * _TILE, _TILE)

            pltpu.sync_copy(x_hbm.at[pl.ds(cbase, _TILE)], idx_v)
            pltpu.async_copy(
                pt_hbm.at[pl.ds(r8, 8), pl.ds(cbase, _TILE)], pos_v, psem
            ).wait()
            fire_group(0, r8, 0)

            def group(g, par, last):
                vec = idx_v[pl.ds(g * _L, _L)]
                lanes = vec & jnp.int32(_TILE - 1)

                @pl.when(jnp.logical_not(last))
                def _():
                    fire_group(g + 1, r8, 1 - par)

                for k in range(_L):
                    pltpu.make_async_copy(
                        wt_hbm.at[pl.ds(0, 8), pl.ds(0, _TILE)],
                        ring_v.at[pl.ds((par * _L + k) * 8, 8), :],
                        sems[par],
                    ).wait()
                for k in range(0, _L, 2):
                    s0 = (par * _L + k) * 8
                    s1 = (par * _L + k + 1) * 8
                    rows = jnp.where(half, s0, s1) + sub
                    cols = jnp.where(half, lanes[k], lanes[k + 1])
                    val = plsc.load_gather(ring_v, [rows, cols])
                    c0 = g * _L + k
                    ocol = jnp.where(half, c0, c0 + 1)
                    pv = plsc.load_gather(pos_v, [sub, ocol])
                    plsc.store_scatter(buf_v, [sub, ocol], val + pv)

            def gpair(m):
                g0 = 2 * m
                group(g0, 0, jnp.bool_(False))
                group(g0 + 1, 1, m + 1 >= ngroups // 2)

            pl.loop(0, ngroups // 2)(gpair)
            pltpu.sync_copy(
                buf_v, out_hbm.at[pl.ds(r8, 8), pl.ds(cbase, _TILE)]
            )

        pl.loop(0, upw)(unit)

    return emb


_S_TC = 2048
_TPS = 8


@functools.lru_cache(maxsize=None)
def _build_tc(s_tc: int, s_base: int, vocab: int, dim: int):
    steps = s_tc // _TPS
    assert s_tc % _TILE == 0 and s_base % _TILE == 0

    spb = _TILE // _TPS

    def body(tc_ref, ln_ref, *refs):
        blocks = refs[:_TPS]
        posb = refs[_TPS]
        out_ref = refs[_TPS + 1]
        i = pl.program_id(0)
        s0 = (i % spb) * _TPS
        ir = lax.broadcasted_iota(jnp.int32, (_TILE, _TILE), 0)
        il = lax.broadcasted_iota(jnp.int32, (_TILE, _TILE), 1)
        placed = jnp.zeros((dim, _TILE), jnp.float32)
        for j in range(_TPS):
            lane_j = ln_ref[i * _TPS + j]
            m = jnp.where((ir == lane_j) & (il == s0 + j), 1.0, 0.0)
            placed = placed + jax.lax.dot_general(
                blocks[j][...], m, (((1,), (0,)), ((), ())),
                preferred_element_type=jnp.float32,
            )
        mp = jnp.where((ir == il) & (il >= s0) & (il < s0 + _TPS), 1.0, 0.0)
        placed = placed + jax.lax.dot_general(
            posb[...], mp, (((1,), (0,)), ((), ())),
            preferred_element_type=jnp.float32,
        )

        @pl.when(i % spb == 0)
        def _():
            out_ref[...] = placed

        @pl.when(i % spb != 0)
        def _():
            out_ref[...] = out_ref[...] + placed

    grid_spec = pltpu.PrefetchScalarGridSpec(
        num_scalar_prefetch=2,
        grid=(steps,),
        in_specs=[
            pl.BlockSpec(
                (dim, _TILE), functools.partial(lambda j, i, tc, ln: (0, tc[i * _TPS + j]), j)
            )
            for j in range(_TPS)
        ]
        + [
            pl.BlockSpec(
                (dim, _TILE),
                lambda i, tc, ln: (0, s_base // _TILE + i // (_TILE // _TPS)),
            )
        ],
        out_specs=pl.BlockSpec(
            (dim, _TILE), lambda i, tc, ln: (0, i // (_TILE // _TPS))
        ),
    )
    return pl.pallas_call(
        body,
        grid_spec=grid_spec,
        out_shape=jax.ShapeDtypeStruct((dim, s_tc), jnp.float32),
        compiler_params=pltpu.CompilerParams(dimension_semantics=("arbitrary",)),
    )


def kernel(x, word_table, pos_table):
    seq_len = x.shape[0]
    vocab, dim = word_table.shape
    xi = x.astype(jnp.int32)
    wt_t = word_table.T
    pos_t = pos_table[:seq_len].T

    s_tc = _S_TC if seq_len > _S_TC else 0
    s_sc = seq_len - s_tc

    emb = _build(s_sc, vocab, dim)
    out_sc = emb(xi, wt_t, pos_t)
    if s_tc == 0:
        return out_sc.T

    x_tc = xi[s_sc:]
    tcols = x_tc >> 7
    lanes = x_tc & (_TILE - 1)
    tc_fn = _build_tc(s_tc, s_sc, vocab, dim)
    out_tc = tc_fn(tcols, lanes, *([wt_t] * _TPS), pos_t)
    return jnp.concatenate([out_sc, out_tc], axis=1).T


# speedup vs baseline: 1.5179x; 1.5179x over previous
import functools

import jax
import jax.numpy as jnp
from jax import lax
from jax.experimental import pallas as pl
from jax.experimental.pallas import tpu as pltpu
from jax.experimental.pallas import tpu_sc as plsc

_L = 16
_TILE = 128
_NBUF = 8
_NHALF = 1


@functools.lru_cache(maxsize=None)
def _build(seq_len: int, vocab: int, dim: int):
    info = plsc.get_sparse_core_info()
    nc, ns = info.num_cores, info.num_subcores
    nw = nc * ns
    assert seq_len % (nw * _L * _NHALF) == 0
    bpw = seq_len // nw
    hpw = bpw // _NHALF
    ngroups = hpw // _L
    assert dim % _L == 0
    nr = dim // _L

    mesh = plsc.VectorSubcoreMesh(core_axis_name="c", subcore_axis_name="s")

    @functools.partial(
        pl.kernel,
        mesh=mesh,
        out_type=jax.ShapeDtypeStruct((dim, seq_len), jnp.float32),
        scratch_types=[
            pltpu.VMEM((bpw,), jnp.int32),
            pltpu.VMEM((_NBUF * dim, _TILE), jnp.float32),
            pltpu.VMEM((dim, hpw), jnp.float32),
            pltpu.VMEM((dim, hpw), jnp.float32),
            [pltpu.SemaphoreType.DMA] * _NBUF,
            pltpu.SemaphoreType.DMA,
        ],
        compiler_params=pltpu.CompilerParams(needs_layout_passes=False),
    )
    def emb(x_hbm, wt_hbm, pt_hbm, out_hbm, idx_v, ring_v, buf_v, pos_v, sems, psem):
        wid = lax.axis_index("s") * nc + lax.axis_index("c")
        base = wid * bpw

        pltpu.sync_copy(x_hbm.at[pl.ds(base, bpw)], idx_v)

        iota = lax.iota(jnp.int32, _L)
        nfire = min(_NBUF, _L)

        def fire(k, t128):
            tk = pl.multiple_of(t128[k], _TILE)
            b = k % _NBUF
            pltpu.async_copy(
                wt_hbm.at[:, pl.ds(tk, _TILE)],
                ring_v.at[pl.ds(b * dim, dim), :],
                sems[b],
            )

        def half(h):
            hbase = base + h * hpw
            pltpu.async_copy(pt_hbm.at[:, pl.ds(hbase, hpw)], pos_v, psem).wait()

            def group(gl):
                vec = idx_v[pl.ds(h * hpw + gl * _L, _L)]
                t128 = vec & jnp.int32(-_TILE)
                lanes = vec & jnp.int32(_TILE - 1)
                for k in range(nfire):
                    fire(k, t128)
                for k in range(_L):
                    b = k % _NBUF
                    pltpu.make_async_copy(
                        wt_hbm.at[:, pl.ds(0, _TILE)],
                        ring_v.at[pl.ds(b * dim, dim), :],
                        sems[b],
                    ).wait()
                    col = jnp.full((_L,), gl * _L + k, jnp.int32)
                    lane = jnp.full((_L,), lanes[k], jnp.int32)
                    for r in range(nr):
                        rows = iota + (b * dim + r * _L)
                        val = plsc.load_gather(ring_v, [rows, lane])
                        prow = iota + r * _L
                        pv = plsc.load_gather(pos_v, [prow, col])
                        plsc.store_scatter(buf_v, [prow, col], val + pv)
                    if k + _NBUF < _L:
                        fire(k + _NBUF, t128)

            pl.loop(0, ngroups)(group)
            pltpu.sync_copy(buf_v, out_hbm.at[:, pl.ds(hbase, hpw)])

        pl.loop(0, _NHALF)(half)

    return emb


def kernel(x, word_table, pos_table):
    seq_len = x.shape[0]
    vocab, dim = word_table.shape
    emb = _build(seq_len, vocab, dim)
    out_t = emb(x.astype(jnp.int32), word_table.T, pos_table[:seq_len].T)
    return out_t.T


# speedup vs baseline: 1.5190x; 1.0008x over previous
import functools

import jax
import jax.numpy as jnp
from jax import lax
from jax.experimental import pallas as pl
from jax.experimental.pallas import tpu as pltpu
from jax.experimental.pallas import tpu_sc as plsc

_L = 16
_TILE = 128
_NBUF = 8
_NHALF = 1


@functools.lru_cache(maxsize=None)
def _build(seq_len: int, vocab: int, dim: int):
    info = plsc.get_sparse_core_info()
    nc, ns = info.num_cores, info.num_subcores
    nw = nc * ns
    assert seq_len % (nw * _L * _NHALF) == 0
    bpw = seq_len // nw
    hpw = bpw // _NHALF
    ngroups = hpw // _L
    assert dim % _L == 0
    nr = dim // _L

    mesh = plsc.VectorSubcoreMesh(core_axis_name="c", subcore_axis_name="s")

    @functools.partial(
        pl.kernel,
        mesh=mesh,
        out_type=jax.ShapeDtypeStruct((dim, seq_len), jnp.float32),
        scratch_types=[
            pltpu.VMEM((bpw,), jnp.int32),
            pltpu.VMEM((_NBUF * dim, _TILE), jnp.float32),
            pltpu.VMEM((dim, hpw), jnp.float32),
            pltpu.VMEM((dim, hpw), jnp.float32),
            [pltpu.SemaphoreType.DMA] * _NBUF,
            pltpu.SemaphoreType.DMA,
        ],
        compiler_params=pltpu.CompilerParams(needs_layout_passes=False),
    )
    def emb(x_hbm, wt_hbm, pt_hbm, out_hbm, idx_v, ring_v, buf_v, pos_v, sems, psem):
        wid = lax.axis_index("s") * nc + lax.axis_index("c")
        base = wid * bpw

        pltpu.sync_copy(x_hbm.at[pl.ds(base, bpw)], idx_v)

        iota = lax.iota(jnp.int32, _L)
        nfire = min(_NBUF, _L)

        def fire(k, t128):
            tk = pl.multiple_of(t128[k], _TILE)
            b = k % _NBUF
            pltpu.async_copy(
                wt_hbm.at[:, pl.ds(tk, _TILE)],
                ring_v.at[pl.ds(b * dim, dim), :],
                sems[b],
            )

        def half(h):
            hbase = base + h * hpw
            pltpu.async_copy(pt_hbm.at[:, pl.ds(hbase, hpw)], pos_v, psem).wait()

            def group(gl):
                vec = idx_v[pl.ds(h * hpw + gl * _L, _L)]
                t128 = vec & jnp.int32(-_TILE)
                lanes = vec & jnp.int32(_TILE - 1)
                for k in range(nfire):
                    fire(k, t128)
                for k in range(_L):
                    b = k % _NBUF
                    pltpu.make_async_copy(
                        wt_hbm.at[:, pl.ds(0, _TILE)],
                        ring_v.at[pl.ds(b * dim, dim), :],
                        sems[b],
                    ).wait()
                    col = jnp.full((_L,), gl * _L + k, jnp.int32)
                    lane = jnp.full((_L,), lanes[k], jnp.int32)
                    for r in range(nr):
                        rows = iota + (b * dim + r * _L)
                        val = plsc.load_gather(ring_v, [rows, lane])
                        prow = iota + r * _L
                        plsc.store_scatter(buf_v, [prow, col], val)
                    if k + _NBUF < _L:
                        fire(k + _NBUF, t128)

            pl.loop(0, ngroups)(group)

            def add_row(c):
                for j in range(hpw // _L):
                    sl = pl.ds(j * _L, _L)
                    buf_v[c, sl] = buf_v[c, sl] + pos_v[c, sl]

            pl.loop(0, dim)(add_row)
            pltpu.sync_copy(buf_v, out_hbm.at[:, pl.ds(hbase, hpw)])

        pl.loop(0, _NHALF)(half)

    return emb


def kernel(x, word_table, pos_table):
    seq_len = x.shape[0]
    vocab, dim = word_table.shape
    emb = _build(seq_len, vocab, dim)
    out_t = emb(x.astype(jnp.int32), word_table.T, pos_table[:seq_len].T)
    return out_t.T
